# Initial kernel scaffold; baseline (speedup 1.0000x reference)
import jax
import jax.numpy as jnp
from jax.experimental import pallas as pl


def kernel(node_feat, coord, vel, virtual_node_feat, virtual_coord, edge_attr, node_attr, edge_index, data_batch, params):
    raise NotImplementedError("write your pallas kernel here")


# speedup vs baseline: 1.1587x; 1.1587x over previous
import functools

import jax
import jax.numpy as jnp
from jax.experimental import pallas as pl


N = 10000
E = 320000
B = 16
NF = 128
HID = 128
C = 3

BE = 4000


def _silu(x):
    return x * jax.nn.sigmoid(x)


def _edge_block_kernel(g_ref, er_ref, w2_ref, b2_ref, w1c_ref, b1c_ref,
                       w2c_ref, w1er_ref, ef_ref, w_ref):
    h1 = g_ref[...] + jnp.dot(er_ref[...], w1er_ref[...],
                              preferred_element_type=jnp.float32)
    h1 = _silu(h1)
    ef = _silu(jnp.dot(h1, w2_ref[...], preferred_element_type=jnp.float32)
               + b2_ref[...])
    ef_ref[...] = ef
    hc = _silu(jnp.dot(ef, w1c_ref[...], preferred_element_type=jnp.float32)
               + b1c_ref[...])
    w = jnp.dot(hc, w2c_ref[...], preferred_element_type=jnp.float32)
    w_ref[...] = w


def _edge_pipeline(G, earad, p_edge, p_coord):
    grid = (E // BE,)
    full = lambda shape: pl.BlockSpec(shape, lambda i: (0, 0))
    ef, w = pl.pallas_call(
        _edge_block_kernel,
        grid=grid,
        in_specs=[
            pl.BlockSpec((BE, HID), lambda i: (i, 0)),
            pl.BlockSpec((BE, 8), lambda i: (i, 0)),
            full((HID, HID)),
            full((1, HID)),
            full((HID, HID)),
            full((1, HID)),
            full((HID, 8)),
            full((8, HID)),
        ],
        out_specs=[
            pl.BlockSpec((BE, HID), lambda i: (i, 0)),
            pl.BlockSpec((BE, 8), lambda i: (i, 0)),
        ],
        out_shape=[
            jax.ShapeDtypeStruct((E, HID), jnp.float32),
            jax.ShapeDtypeStruct((E, 8), jnp.float32),
        ],
    )(G, earad,
      p_edge['W2'], p_edge['b2'][None, :],
      p_coord['W1'], p_coord['b1'][None, :],
      jnp.pad(p_coord['W2'], ((0, 0), (0, 7))),
      jnp.zeros((8, HID), jnp.float32).at[:5].set(
          jnp.concatenate([p_edge['W1'][2 * NF + 1:],
                           p_edge['W1'][2 * NF:2 * NF + 1]], axis=0)))
    return ef, w[:, :1]


def kernel(node_feat, coord, vel, virtual_node_feat, virtual_coord,
           edge_attr, node_attr, edge_index, data_batch, params):
    p = params
    row, col = edge_index[0], edge_index[1]

    W1 = p['edge_mlp']['W1']
    Pa = node_feat @ W1[:NF]
    Pb = node_feat @ W1[NF:2 * NF] + p['edge_mlp']['b1']

    cd = coord[row] - coord[col]
    radial = jnp.sum(cd * cd, axis=1, keepdims=True)
    earad = jnp.concatenate(
        [edge_attr, radial, jnp.zeros((E, 3), jnp.float32)], axis=1)
    G = Pa[row] + Pb[col]

    ef, w = _edge_pipeline(G, earad, p['edge_mlp'], p['coord_mlp_r'])

    trans = cd * w

    cnt = jax.ops.segment_sum(jnp.ones((E,), jnp.float32), row, num_segments=N)
    inv = 1.0 / jnp.maximum(cnt, 1.0)
    agg = jax.ops.segment_sum(trans, row, num_segments=N) * inv[:, None]
    agg_e = jax.ops.segment_sum(ef, row, num_segments=N) * inv[:, None]

    vcoord_b = virtual_coord[data_batch]
    v_coord_diff = coord[:, :, None] - vcoord_b
    v_radial = jnp.sum(v_coord_diff ** 2, axis=1, keepdims=True)
    mix_V = jnp.einsum('bdi,bdj->bij', virtual_coord, virtual_coord)[data_batch]

    pv = p['edge_mlp_virtual']
    W1v = pv['W1']
    h1v = (node_feat @ W1v[:NF])[:, None, :]
    vproj = jnp.einsum('bfc,fh->bch', virtual_node_feat, W1v[NF:2 * NF])
    h1v = h1v + vproj[data_batch]
    h1v = h1v + v_radial[:, 0, :, None] * W1v[2 * NF][None, None, :]
    h1v = h1v + jnp.einsum('ncj,jh->nch', mix_V, W1v[2 * NF + 1:])
    h1v = _silu(h1v + pv['b1'])
    vef = _silu(jnp.einsum('nch,hk->nck', h1v, pv['W2']) + pv['b2'])

    def coord_head(x, pc):
        h = _silu(jnp.einsum('nch,hk->nck', x, pc['W1']) + pc['b1'])
        return jnp.einsum('nck,ko->nco', h, pc['W2'])

    w_r_v = coord_head(vef, p['coord_mlp_r_virtual'])[..., 0]
    trans_v = jnp.mean(-v_coord_diff * w_r_v[:, None, :], axis=-1)

    w_v_v = coord_head(vef, p['coord_mlp_v_virtual'])[..., 0]
    trans2 = v_coord_diff * w_v_v[:, None, :]

    bcnt = jax.ops.segment_sum(jnp.ones((N,), jnp.float32), data_batch,
                               num_segments=B)
    binv = 1.0 / jnp.maximum(bcnt, 1.0)
    aggv = jax.ops.segment_sum(trans2.reshape(N, -1), data_batch,
                               num_segments=B) * binv[:, None]
    virtual_coord_out = virtual_coord + aggv.reshape(B, 3, C)

    pcv = p['coord_mlp_vel']
    velw = _silu(node_feat @ pcv['W1'] + pcv['b1']) @ pcv['W2'] + pcv['b2']
    coord_out = coord + agg + trans_v + velw * vel

    agg_v = jnp.mean(vef, axis=1)
    n_in = jnp.concatenate([node_feat, agg_e, agg_v, node_attr], axis=1)
    pn = p['node_mlp']
    node_out = node_feat + (_silu(n_in @ pn['W1'] + pn['b1']) @ pn['W2']
                            + pn['b2'])

    aggvn = (jax.ops.segment_sum(vef.reshape(N, -1), data_batch,
                                 num_segments=B) * binv[:, None]).reshape(B, C, HID)
    vn_in = jnp.concatenate([jnp.swapaxes(virtual_node_feat, 1, 2), aggvn],
                            axis=2)
    pvn = p['node_mlp_virtual']
    vn_out = virtual_node_feat + jnp.swapaxes(
        _silu(vn_in @ pvn['W1'] + pvn['b1']) @ pvn['W2'] + pvn['b2'], 1, 2)

    return (node_out, coord_out, vn_out, virtual_coord_out)


# speedup vs baseline: 1.2481x; 1.0771x over previous
import functools

import jax
import jax.numpy as jnp
from jax.experimental import pallas as pl


N = 10000
E = 320000
B = 16
NF = 128
HID = 128
C = 3

BE = 4000


def _silu(x):
    return x * jax.nn.sigmoid(x)


def _edge_block_kernel(g_ref, er_ref, w2_ref, b2_ref, w1c_ref, b1c_ref,
                       w2c_ref, w1er_ref, ef_ref, w_ref):
    h1 = g_ref[...] + jnp.dot(er_ref[...], w1er_ref[...],
                              preferred_element_type=jnp.float32)
    h1 = _silu(h1)
    ef = _silu(jnp.dot(h1, w2_ref[...], preferred_element_type=jnp.float32)
               + b2_ref[...])
    ef_ref[...] = ef
    hc = _silu(jnp.dot(ef, w1c_ref[...], preferred_element_type=jnp.float32)
               + b1c_ref[...])
    w = jnp.dot(hc, w2c_ref[...], preferred_element_type=jnp.float32)
    w_ref[...] = w


def _edge_pipeline(G, earad, p_edge, p_coord):
    grid = (E // BE,)
    full = lambda shape: pl.BlockSpec(shape, lambda i: (0, 0))
    ef, w = pl.pallas_call(
        _edge_block_kernel,
        grid=grid,
        in_specs=[
            pl.BlockSpec((BE, HID), lambda i: (i, 0)),
            pl.BlockSpec((BE, 8), lambda i: (i, 0)),
            full((HID, HID)),
            full((1, HID)),
            full((HID, HID)),
            full((1, HID)),
            full((HID, 8)),
            full((8, HID)),
        ],
        out_specs=[
            pl.BlockSpec((BE, HID), lambda i: (i, 0)),
            pl.BlockSpec((BE, 8), lambda i: (i, 0)),
        ],
        out_shape=[
            jax.ShapeDtypeStruct((E, HID), jnp.float32),
            jax.ShapeDtypeStruct((E, 8), jnp.float32),
        ],
    )(G, earad,
      p_edge['W2'], p_edge['b2'][None, :],
      p_coord['W1'], p_coord['b1'][None, :],
      jnp.pad(p_coord['W2'], ((0, 0), (0, 7))),
      jnp.zeros((8, HID), jnp.float32).at[:5].set(
          jnp.concatenate([p_edge['W1'][2 * NF + 1:],
                           p_edge['W1'][2 * NF:2 * NF + 1]], axis=0)))
    return ef, w[:, :1]


def kernel(node_feat, coord, vel, virtual_node_feat, virtual_coord,
           edge_attr, node_attr, edge_index, data_batch, params):
    p = params
    row, col = edge_index[0], edge_index[1]

    W1 = p['edge_mlp']['W1']
    Pa = node_feat @ W1[:NF]
    Pb = node_feat @ W1[NF:2 * NF] + p['edge_mlp']['b1']

    cd = coord[row] - coord[col]
    radial = jnp.sum(cd * cd, axis=1, keepdims=True)
    earad = jnp.concatenate(
        [edge_attr, radial, jnp.zeros((E, 3), jnp.float32)], axis=1)
    G = Pa[row] + Pb[col]

    ef, w = _edge_pipeline(G, earad, p['edge_mlp'], p['coord_mlp_r'])

    trans = cd * w

    cnt = jax.ops.segment_sum(jnp.ones((E,), jnp.float32), row, num_segments=N)
    inv = 1.0 / jnp.maximum(cnt, 1.0)
    agg = jax.ops.segment_sum(trans, row, num_segments=N) * inv[:, None]
    agg_e = jax.ops.segment_sum(ef, row, num_segments=N) * inv[:, None]

    vcoord_b = virtual_coord[data_batch]
    v_coord_diff = coord[:, :, None] - vcoord_b
    v_radial = jnp.sum(v_coord_diff ** 2, axis=1, keepdims=True)
    mix_V = jnp.einsum('bdi,bdj->bij', virtual_coord, virtual_coord)[data_batch]

    pv = p['edge_mlp_virtual']
    W1v = pv['W1']
    h1v = (node_feat @ W1v[:NF])[:, None, :]
    vproj = jnp.einsum('bfc,fh->bch', virtual_node_feat, W1v[NF:2 * NF])
    h1v = h1v + vproj[data_batch]
    h1v = h1v + v_radial[:, 0, :, None] * W1v[2 * NF][None, None, :]
    h1v = h1v + jnp.einsum('ncj,jh->nch', mix_V, W1v[2 * NF + 1:])
    h1v = _silu(h1v + pv['b1'])
    vef = _silu(jnp.einsum('nch,hk->nck', h1v, pv['W2']) + pv['b2'])

    def coord_head(x, pc):
        h = _silu(jnp.einsum('nch,hk->nck', x, pc['W1']) + pc['b1'])
        return jnp.einsum('nck,ko->nco', h, pc['W2'])

    w_r_v = coord_head(vef, p['coord_mlp_r_virtual'])[..., 0]
    trans_v = jnp.mean(-v_coord_diff * w_r_v[:, None, :], axis=-1)

    w_v_v = coord_head(vef, p['coord_mlp_v_virtual'])[..., 0]
    trans2 = v_coord_diff * w_v_v[:, None, :]

    onehot = (data_batch[:, None] == jnp.arange(B)[None, :]).astype(jnp.float32)
    bcnt = jnp.sum(onehot, axis=0)
    binv = 1.0 / jnp.maximum(bcnt, 1.0)
    aggv = onehot.T @ trans2.reshape(N, -1) * binv[:, None]
    virtual_coord_out = virtual_coord + aggv.reshape(B, 3, C)

    pcv = p['coord_mlp_vel']
    velw = _silu(node_feat @ pcv['W1'] + pcv['b1']) @ pcv['W2'] + pcv['b2']
    coord_out = coord + agg + trans_v + velw * vel

    agg_v = jnp.mean(vef, axis=1)
    n_in = jnp.concatenate([node_feat, agg_e, agg_v, node_attr], axis=1)
    pn = p['node_mlp']
    node_out = node_feat + (_silu(n_in @ pn['W1'] + pn['b1']) @ pn['W2']
                            + pn['b2'])

    aggvn = (onehot.T @ vef.reshape(N, -1) * binv[:, None]).reshape(B, C, HID)
    vn_in = jnp.concatenate([jnp.swapaxes(virtual_node_feat, 1, 2), aggvn],
                            axis=2)
    pvn = p['node_mlp_virtual']
    vn_out = virtual_node_feat + jnp.swapaxes(
        _silu(vn_in @ pvn['W1'] + pvn['b1']) @ pvn['W2'] + pvn['b2'], 1, 2)

    return (node_out, coord_out, vn_out, virtual_coord_out)


# speedup vs baseline: 1.7721x; 1.4199x over previous
import functools

import jax
import jax.numpy as jnp
from jax import lax
from jax.experimental import pallas as pl
from jax.experimental.pallas import tpu as pltpu
from jax.experimental.pallas import tpu_sc as plsc


N = 10000
NP = 10240
E = 320000
B = 16
NF = 128
HID = 128
C = 3

BE = 4000
CH = 128
NCHUNK = E // CH
NTILE = NP // 16

_MESH = plsc.VectorSubcoreMesh(core_axis_name="c", subcore_axis_name="s")


def _silu(x):
    return x * jax.nn.sigmoid(x)


def _edge_block_kernel(g_ref, er_ref, cdr_ref, w2_ref, b2_ref, w1c_ref,
                       b1c_ref, w2c_ref, w1er_ref, ef_ref, cdw_ref):
    h1 = g_ref[...] + jnp.dot(er_ref[...], w1er_ref[...],
                              preferred_element_type=jnp.float32)
    h1 = _silu(h1)
    ef = _silu(jnp.dot(h1, w2_ref[...], preferred_element_type=jnp.float32)
               + b2_ref[...])
    ef_ref[...] = ef
    hc = _silu(jnp.dot(ef, w1c_ref[...], preferred_element_type=jnp.float32)
               + b1c_ref[...])
    w = jnp.dot(hc, w2c_ref[...], preferred_element_type=jnp.float32)
    cdw_ref[...] = jnp.concatenate(
        [cdr_ref[:, :3], w[:, :1], jnp.zeros((BE, 4), jnp.float32)], axis=1)


def _edge_pipeline(G, earad, cdr, p_edge, p_coord):
    grid = (E // BE,)
    full = lambda shape: pl.BlockSpec(shape, lambda i: (0, 0))
    ef, cdw = pl.pallas_call(
        _edge_block_kernel,
        grid=grid,
        in_specs=[
            pl.BlockSpec((BE, HID), lambda i: (i, 0)),
            pl.BlockSpec((BE, 8), lambda i: (i, 0)),
            pl.BlockSpec((BE, 8), lambda i: (i, 0)),
            full((HID, HID)),
            full((1, HID)),
            full((HID, HID)),
            full((1, HID)),
            full((HID, 8)),
            full((8, HID)),
        ],
        out_specs=[
            pl.BlockSpec((BE, HID), lambda i: (i, 0)),
            pl.BlockSpec((BE, 8), lambda i: (i, 0)),
        ],
        out_shape=[
            jax.ShapeDtypeStruct((E, HID), jnp.float32),
            jax.ShapeDtypeStruct((E, 8), jnp.float32),
        ],
    )(G, earad, cdr,
      p_edge['W2'], p_edge['b2'][None, :],
      p_coord['W1'], p_coord['b1'][None, :],
      jnp.pad(p_coord['W2'], ((0, 0), (0, 7))),
      jnp.zeros((8, HID), jnp.float32).at[:5].set(
          jnp.concatenate([p_edge['W1'][2 * NF + 1:],
                           p_edge['W1'][2 * NF:2 * NF + 1]], axis=0)))
    return ef, cdw


@functools.partial(
    pl.kernel, mesh=_MESH,
    out_type=jax.ShapeDtypeStruct((2, NP, HID), jnp.float32),
    scratch_types=[
        pltpu.VMEM((CH,), jnp.int32),
        pltpu.VMEM((CH, HID), jnp.float32),
        pltpu.VMEM_SHARED((NP, HID), jnp.float32),
    ],
)
def _sc_scatter_ef(idx_hbm, ef_hbm, zero_hbm, out_hbm, idx_v, ef_v, acc):
    cid = lax.axis_index("c")
    sid = lax.axis_index("s")
    wid = sid * 2 + cid
    pltpu.sync_copy(zero_hbm.at[pl.ds(sid * NTILE, NTILE)],
                    acc.at[pl.ds(sid * NTILE, NTILE)])
    plsc.subcore_barrier()
    n_my = (NCHUNK - wid + 31) // 32

    def body(kk, carry):
        c = wid + kk * 32
        pltpu.sync_copy(idx_hbm.at[c], idx_v)
        pltpu.sync_copy(ef_hbm.at[pl.ds(c * CH, CH)], ef_v)
        pltpu.sync_copy(ef_v, acc.at[idx_v], add=True)
        return carry

    lax.fori_loop(0, n_my, body, 0)
    plsc.subcore_barrier()
    pltpu.sync_copy(acc.at[pl.ds(sid * NTILE, NTILE)],
                    out_hbm.at[cid, pl.ds(sid * NTILE, NTILE)])


@functools.partial(
    pl.kernel, mesh=_MESH,
    compiler_params=pltpu.CompilerParams(needs_layout_passes=False),
    out_type=jax.ShapeDtypeStruct((32, NP * 4), jnp.float32),
    scratch_types=[
        pltpu.VMEM((CH,), jnp.int32),
        pltpu.VMEM((CH, 8), jnp.float32),
        pltpu.VMEM((NP * 4,), jnp.float32),
    ],
)
def _sc_scatter_tail(idx_hbm, cdw_hbm, zero_hbm, out_hbm, idx_v, cdw_v, acct):
    cid = lax.axis_index("c")
    sid = lax.axis_index("s")
    wid = sid * 2 + cid
    pltpu.sync_copy(zero_hbm, acct)
    n_my = (NCHUNK - wid + 31) // 32
    lan = lax.iota(jnp.int32, 16)
    c0 = jnp.zeros((16,), jnp.int32)
    c1 = c0 + 1
    c2 = c0 + 2
    c3 = c0 + 3
    fones = jnp.ones((16,), jnp.float32)

    def body(kk, carry):
        c = wid + kk * 32
        pltpu.sync_copy(idx_hbm.at[c], idx_v)
        pltpu.sync_copy(cdw_hbm.at[pl.ds(c * CH, CH)], cdw_v)
        for g in range(CH // 16):
            rows4 = idx_v[pl.ds(g * 16, 16)] * 4
            e16 = g * 16 + lan
            cdx = plsc.load_gather(cdw_v, [e16, c0])
            cdy = plsc.load_gather(cdw_v, [e16, c1])
            cdz = plsc.load_gather(cdw_v, [e16, c2])
            w16 = plsc.load_gather(cdw_v, [e16, c3])
            plsc.addupdate_scatter(acct, [rows4], cdx * w16)
            plsc.addupdate_scatter(acct, [rows4 + 1], cdy * w16)
            plsc.addupdate_scatter(acct, [rows4 + 2], cdz * w16)
            plsc.addupdate_scatter(acct, [rows4 + 3], fones)
        return carry

    lax.fori_loop(0, n_my, body, 0)
    pltpu.sync_copy(acct, out_hbm.at[wid])


def kernel(node_feat, coord, vel, virtual_node_feat, virtual_coord,
           edge_attr, node_attr, edge_index, data_batch, params):
    p = params
    row = edge_index[0].astype(jnp.int32)
    col = edge_index[1].astype(jnp.int32)

    W1 = p['edge_mlp']['W1']
    Pa = node_feat @ W1[:NF]
    Pb = node_feat @ W1[NF:2 * NF] + p['edge_mlp']['b1']

    cd = coord[row] - coord[col]
    radial = jnp.sum(cd * cd, axis=1, keepdims=True)
    earad = jnp.concatenate(
        [edge_attr, radial, jnp.zeros((E, 3), jnp.float32)], axis=1)
    cdr = jnp.concatenate([cd, jnp.zeros((E, 5), jnp.float32)], axis=1)
    G = Pa[row] + Pb[col]

    ef, cdw = _edge_pipeline(G, earad, cdr, p['edge_mlp'], p['coord_mlp_r'])

    idx2 = row.reshape(NCHUNK, CH)
    outef = _sc_scatter_ef(idx2, ef, jnp.zeros((NP, HID), jnp.float32))
    outtail = _sc_scatter_tail(idx2, cdw, jnp.zeros((NP * 4,), jnp.float32))
    sef = (outef[0] + outef[1])[:N]
    tail = outtail.sum(axis=0).reshape(NP, 4)[:N]
    inv = 1.0 / jnp.maximum(tail[:, 3], 1.0)
    agg = tail[:, :3] * inv[:, None]
    agg_e = sef * inv[:, None]

    vcoord_b = virtual_coord[data_batch]
    v_coord_diff = coord[:, :, None] - vcoord_b
    v_radial = jnp.sum(v_coord_diff ** 2, axis=1, keepdims=True)
    mix_V = jnp.einsum('bdi,bdj->bij', virtual_coord, virtual_coord)[data_batch]

    pv = p['edge_mlp_virtual']
    W1v = pv['W1']
    h1v = (node_feat @ W1v[:NF])[:, None, :]
    vproj = jnp.einsum('bfc,fh->bch', virtual_node_feat, W1v[NF:2 * NF])
    h1v = h1v + vproj[data_batch]
    h1v = h1v + v_radial[:, 0, :, None] * W1v[2 * NF][None, None, :]
    h1v = h1v + jnp.einsum('ncj,jh->nch', mix_V, W1v[2 * NF + 1:])
    h1v = _silu(h1v + pv['b1'])
    vef = _silu(jnp.einsum('nch,hk->nck', h1v, pv['W2']) + pv['b2'])

    def coord_head(x, pc):
        h = _silu(jnp.einsum('nch,hk->nck', x, pc['W1']) + pc['b1'])
        return jnp.einsum('nck,ko->nco', h, pc['W2'])

    w_r_v = coord_head(vef, p['coord_mlp_r_virtual'])[..., 0]
    trans_v = jnp.mean(-v_coord_diff * w_r_v[:, None, :], axis=-1)

    w_v_v = coord_head(vef, p['coord_mlp_v_virtual'])[..., 0]
    trans2 = v_coord_diff * w_v_v[:, None, :]

    onehot = (data_batch[:, None] == jnp.arange(B)[None, :]).astype(jnp.float32)
    bcnt = jnp.sum(onehot, axis=0)
    binv = 1.0 / jnp.maximum(bcnt, 1.0)
    aggv = onehot.T @ trans2.reshape(N, -1) * binv[:, None]
    virtual_coord_out = virtual_coord + aggv.reshape(B, 3, C)

    pcv = p['coord_mlp_vel']
    velw = _silu(node_feat @ pcv['W1'] + pcv['b1']) @ pcv['W2'] + pcv['b2']
    coord_out = coord + agg + trans_v + velw * vel

    agg_v = jnp.mean(vef, axis=1)
    n_in = jnp.concatenate([node_feat, agg_e, agg_v, node_attr], axis=1)
    pn = p['node_mlp']
    node_out = node_feat + (_silu(n_in @ pn['W1'] + pn['b1']) @ pn['W2']
                            + pn['b2'])

    aggvn = (onehot.T @ vef.reshape(N, -1) * binv[:, None]).reshape(B, C, HID)
    vn_in = jnp.concatenate([jnp.swapaxes(virtual_node_feat, 1, 2), aggvn],
                            axis=2)
    pvn = p['node_mlp_virtual']
    vn_out = virtual_node_feat + jnp.swapaxes(
        _silu(vn_in @ pvn['W1'] + pvn['b1']) @ pvn['W2'] + pvn['b2'], 1, 2)

    return (node_out, coord_out, vn_out, virtual_coord_out)
